# Initial kernel scaffold; baseline (speedup 1.0000x reference)
import jax
import jax.numpy as jnp
from jax.experimental import pallas as pl


def kernel(x, table):
    raise NotImplementedError("write your pallas kernel here")


# speedup vs baseline: 1.2610x; 1.2610x over previous
import functools
import math

import jax
import jax.numpy as jnp
from jax import lax
from jax.experimental import pallas as pl
from jax.experimental.pallas import tpu as pltpu
from jax.experimental.pallas import tpu_sc as plsc

_D = 512
_SCALE = math.sqrt(512.0)
_CHUNK = 128
_LANES = 16


def _emb_body(idx_hbm, table_hbm, out_hbm, idx_v, rows_v, sem, *, b_per_w, nc):
    wid = lax.axis_index("s") * nc + lax.axis_index("c")
    base = wid * b_per_w
    n_chunks = b_per_w // _CHUNK

    def chunk_body(ci, carry):
        off = base + ci * _CHUNK
        pltpu.sync_copy(idx_hbm.at[pl.ds(off, _CHUNK)], idx_v)
        pltpu.async_copy(table_hbm.at[idx_v], rows_v, sem).wait()

        def row_body(r, c2):
            for j in range(_D // _LANES):
                sl = pl.ds(j * _LANES, _LANES)
                rows_v[r, sl] = rows_v[r, sl] * _SCALE
            return c2

        lax.fori_loop(0, _CHUNK, row_body, 0)
        pltpu.sync_copy(rows_v, out_hbm.at[pl.ds(off, _CHUNK)])
        return carry

    lax.fori_loop(0, n_chunks, chunk_body, 0)


def kernel(x, table):
    rows, cols = x.shape
    b_total = rows * cols
    info = plsc.get_sparse_core_info()
    nc, ns = info.num_cores, info.num_subcores
    nw = nc * ns
    b_per_w = b_total // nw

    mesh = plsc.VectorSubcoreMesh(core_axis_name="c", subcore_axis_name="s")
    body = functools.partial(_emb_body, b_per_w=b_per_w, nc=nc)
    run = pl.kernel(
        body,
        mesh=mesh,
        out_type=jax.ShapeDtypeStruct((b_total, _D), jnp.float32),
        scratch_types=[
            pltpu.VMEM((_CHUNK,), jnp.int32),
            pltpu.VMEM((_CHUNK, _D), jnp.float32),
            pltpu.SemaphoreType.DMA,
        ],
    )
    idx = x.reshape(-1).astype(jnp.int32)
    out = run(idx, table)
    return out.reshape(rows, cols, _D)


# speedup vs baseline: 1.8498x; 1.4670x over previous
import functools
import math

import jax
import jax.numpy as jnp
from jax import lax
from jax.experimental import pallas as pl
from jax.experimental.pallas import tpu as pltpu
from jax.experimental.pallas import tpu_sc as plsc

_D = 512
_SCALE = math.sqrt(512.0)
_CHUNK = 40
_NBUF = 4
_LEAD = 2
_LANES = 16


def _emb_body(idx_hbm, table_hbm, out_hbm, *refs, b_per_w, nc):
    idx_v = refs[0]
    bufs = refs[1:1 + _NBUF]
    sem_g = refs[1 + _NBUF:1 + 2 * _NBUF]
    sem_s = refs[1 + 2 * _NBUF:1 + 3 * _NBUF]

    wid = lax.axis_index("s") * nc + lax.axis_index("c")
    base = wid * b_per_w
    n_chunks = b_per_w // _CHUNK

    def gather(ci, slot):
        isl = idx_v.at[pl.ds(ci * _CHUNK, _CHUNK)]
        return pltpu.make_async_copy(table_hbm.at[isl], bufs[slot], sem_g[slot])

    def store(ci, slot):
        osl = out_hbm.at[pl.ds(base + ci * _CHUNK, _CHUNK)]
        return pltpu.make_async_copy(bufs[slot], osl, sem_s[slot])

    pltpu.sync_copy(idx_hbm.at[pl.ds(base, b_per_w)], idx_v)

    for g in range(_LEAD):
        gather(g, g).start()

    def step(s, carry):
        for b in range(_NBUF):
            ci = s * _NBUF + b
            slot_next = (b + _LEAD) % _NBUF

            @pl.when(ci + _LEAD < n_chunks)
            def _issue():
                @pl.when(ci >= _NBUF - _LEAD)
                def _drain_store():
                    store(ci, slot_next).wait()
                gather(ci + _LEAD, slot_next).start()

            gather(ci, b).wait()
            buf = bufs[b]

            def row_body(r, c2):
                for j in range(_D // _LANES):
                    sl = pl.ds(j * _LANES, _LANES)
                    buf[r, sl] = buf[r, sl] * _SCALE
                return c2

            lax.fori_loop(0, _CHUNK, row_body, 0)
            store(ci, b).start()
        return carry

    lax.fori_loop(0, n_chunks // _NBUF, step, 0)

    for b in range(_NBUF):
        store(0, b).wait()


def kernel(x, table):
    rows, cols = x.shape
    b_total = rows * cols
    info = plsc.get_sparse_core_info()
    nc, ns = info.num_cores, info.num_subcores
    nw = nc * ns
    b_per_w = b_total // nw

    mesh = plsc.VectorSubcoreMesh(core_axis_name="c", subcore_axis_name="s")
    body = functools.partial(_emb_body, b_per_w=b_per_w, nc=nc)
    run = pl.kernel(
        body,
        mesh=mesh,
        out_type=jax.ShapeDtypeStruct((b_total, _D), jnp.float32),
        scratch_types=(
            [pltpu.VMEM((b_per_w,), jnp.int32)]
            + [pltpu.VMEM((_CHUNK, _D), jnp.float32) for _ in range(_NBUF)]
            + [pltpu.SemaphoreType.DMA for _ in range(2 * _NBUF)]
        ),
    )
    idx = x.reshape(-1).astype(jnp.int32)
    out = run(idx, table)
    return out.reshape(rows, cols, _D)


# speedup vs baseline: 1.8517x; 1.0010x over previous
import functools
import math

import jax
import jax.numpy as jnp
from jax import lax
from jax.experimental import pallas as pl
from jax.experimental.pallas import tpu as pltpu
from jax.experimental.pallas import tpu_sc as plsc

_D = 512
_SCALE = math.sqrt(512.0)
_CHUNK = 40
_NBUF = 5
_LEAD = 3
_LANES = 16


def _emb_body(idx_hbm, table_hbm, out_hbm, *refs, b_per_w, nc):
    idx_v = refs[0]
    bufs = refs[1:1 + _NBUF]
    sem_g = refs[1 + _NBUF:1 + 2 * _NBUF]
    sem_s = refs[1 + 2 * _NBUF:1 + 3 * _NBUF]

    wid = lax.axis_index("s") * nc + lax.axis_index("c")
    base = wid * b_per_w
    n_chunks = b_per_w // _CHUNK

    def gather(ci, slot):
        isl = idx_v.at[pl.ds(ci * _CHUNK, _CHUNK)]
        return pltpu.make_async_copy(table_hbm.at[isl], bufs[slot], sem_g[slot])

    def store(ci, slot):
        osl = out_hbm.at[pl.ds(base + ci * _CHUNK, _CHUNK)]
        return pltpu.make_async_copy(bufs[slot], osl, sem_s[slot])

    pltpu.sync_copy(idx_hbm.at[pl.ds(base, b_per_w)], idx_v)

    for g in range(_LEAD):
        gather(g, g).start()

    def step(s, carry):
        for b in range(_NBUF):
            ci = s * _NBUF + b
            slot_next = (b + _LEAD) % _NBUF

            @pl.when(ci + _LEAD < n_chunks)
            def _issue():
                @pl.when(ci >= _NBUF - _LEAD)
                def _drain_store():
                    store(ci, slot_next).wait()
                gather(ci + _LEAD, slot_next).start()

            gather(ci, b).wait()
            buf = bufs[b]

            def row_body(r, c2):
                for j in range(_D // _LANES):
                    sl = pl.ds(j * _LANES, _LANES)
                    buf[r, sl] = buf[r, sl] * _SCALE
                return c2

            lax.fori_loop(0, _CHUNK, row_body, 0)
            store(ci, b).start()
        return carry

    lax.fori_loop(0, n_chunks // _NBUF, step, 0)

    for b in range(_NBUF):
        store(0, b).wait()


def kernel(x, table):
    rows, cols = x.shape
    b_total = rows * cols
    info = plsc.get_sparse_core_info()
    nc, ns = info.num_cores, info.num_subcores
    nw = nc * ns
    b_per_w = b_total // nw

    mesh = plsc.VectorSubcoreMesh(core_axis_name="c", subcore_axis_name="s")
    body = functools.partial(_emb_body, b_per_w=b_per_w, nc=nc)
    run = pl.kernel(
        body,
        mesh=mesh,
        out_type=jax.ShapeDtypeStruct((b_total, _D), jnp.float32),
        scratch_types=(
            [pltpu.VMEM((b_per_w,), jnp.int32)]
            + [pltpu.VMEM((_CHUNK, _D), jnp.float32) for _ in range(_NBUF)]
            + [pltpu.SemaphoreType.DMA for _ in range(2 * _NBUF)]
        ),
    )
    idx = x.reshape(-1).astype(jnp.int32)
    out = run(idx, table)
    return out.reshape(rows, cols, _D)
